# baseline (device time: 50476 ns/iter reference)
import functools

import jax
import jax.numpy as jnp
from jax import lax
from jax.experimental import pallas as pl
from jax.experimental.pallas import tpu as pltpu

T = 2048
D = 1024
NQ = 4
TQ = T // NQ
NC = 4
CH = TQ // NC

_MESH = pl.DeviceIdType.MESH


def kernel(ids, E):
    Vs = E.shape[0]

    x = lax.axis_index("x")
    y = lax.axis_index("y")
    z = lax.axis_index("z")
    q = 2 * x + z
    vlo = y * Vs

    my_ids = lax.dynamic_slice(ids, (q * TQ,), (TQ,))
    in_range = jnp.logical_and(my_ids >= vlo, my_ids < vlo + Vs)
    lids = jnp.where(in_range, my_ids - vlo, 0).astype(jnp.int32)
    mask = in_range.astype(jnp.float32).reshape(TQ, 1)

    def body(lids_smem, mask_vmem, e_hbm, out_ref,
             gbuf, acc, ybuf, gsem, ys, yr, xs, xr, zs, zr):
        x = lax.axis_index("x")
        y = lax.axis_index("y")
        z = lax.axis_index("z")
        q = 2 * x + z
        qx = 2 * (1 - x) + z

        nbr_x = (1 - x, y, z)
        nbr_y = (x, 1 - y, z)
        nbr_z = (x, y, 1 - z)

        bar = pltpu.get_barrier_semaphore()
        for nbr in (nbr_x, nbr_y, nbr_z):
            pl.semaphore_signal(bar, inc=1, device_id=nbr,
                                device_id_type=_MESH)
        pl.semaphore_wait(bar, 3)

        def issue(t, _):
            pltpu.make_async_copy(e_hbm.at[pl.ds(lids_smem[t], 1)],
                                  gbuf.at[pl.ds(t, 1)], gsem).start()
            return 0
        lax.fori_loop(0, TQ, issue, 0)

        def drain_rows(n):
            def w(t, _):
                pltpu.make_async_copy(e_hbm.at[pl.ds(0, 1)],
                                      gbuf.at[pl.ds(0, 1)], gsem).wait()
                return 0
            lax.fori_loop(0, n, w, 0)

        def rows(c):
            return pl.ds(c * CH, CH)

        def y_rdma(c):
            return pltpu.make_async_remote_copy(
                src_ref=acc.at[q, rows(c)], dst_ref=ybuf.at[rows(c)],
                send_sem=ys.at[c], recv_sem=yr.at[c],
                device_id=nbr_y, device_id_type=_MESH)

        def x_rdma(c):
            return pltpu.make_async_remote_copy(
                src_ref=acc.at[q, rows(c)], dst_ref=acc.at[q, rows(c)],
                send_sem=xs.at[c], recv_sem=xr.at[c],
                device_id=nbr_x, device_id_type=_MESH)

        def z_rdma(slot, c, si):
            return pltpu.make_async_remote_copy(
                src_ref=acc.at[slot, rows(c)], dst_ref=acc.at[slot, rows(c)],
                send_sem=zs.at[si], recv_sem=zr.at[si],
                device_id=nbr_z, device_id_type=_MESH)

        for c in range(NC):
            drain_rows(CH)
            part = (gbuf[rows(c)] * mask_vmem[rows(c)]).astype(jnp.bfloat16)
            acc[pl.ds(q, 1), rows(c)] = part[None]
            y_rdma(c).start()

        for c in range(NC):
            y_rdma(c).wait_recv()
            acc[pl.ds(q, 1), rows(c)] = (
                acc[pl.ds(q, 1), rows(c)] + ybuf[rows(c)][None])
            x_rdma(c).start()
            z_rdma(q, c, c).start()

        for c in range(NC):
            x_rdma(c).wait_recv()
            z_rdma(qx, c, NC + c).start()

        out_ref[pl.ds(q * TQ, TQ), :] = acc[pl.ds(q, 1)][0].astype(jnp.float32)
        out_ref[pl.ds(qx * TQ, TQ), :] = acc[pl.ds(qx, 1)][0].astype(
            jnp.float32)

        for si in range(2 * NC):
            z_rdma(1 - z, 0, si).wait_recv()
        zq = 1 - z
        zq2 = 3 - z
        out_ref[pl.ds(zq * TQ, TQ), :] = acc[pl.ds(zq, 1)][0].astype(
            jnp.float32)
        out_ref[pl.ds(zq2 * TQ, TQ), :] = acc[pl.ds(zq2, 1)][0].astype(
            jnp.float32)

        for c in range(NC):
            y_rdma(c).wait_send()
            x_rdma(c).wait_send()
        for si in range(2 * NC):
            z_rdma(z, 0, si).wait_send()

        @functools.partial(pl.run_scoped, sem2=pltpu.SemaphoreType.REGULAR)
        def _(sem2):
            for nbr in (nbr_x, nbr_y, nbr_z):
                pl.semaphore_signal(sem2, inc=1, device_id=nbr,
                                    device_id_type=_MESH)
            pl.semaphore_wait(sem2, 3)

    return pl.pallas_call(
        body,
        out_shape=jax.ShapeDtypeStruct((T, D), jnp.float32),
        in_specs=[
            pl.BlockSpec(memory_space=pltpu.SMEM),
            pl.BlockSpec(memory_space=pltpu.VMEM),
            pl.BlockSpec(memory_space=pltpu.MemorySpace.HBM),
        ],
        out_specs=pl.BlockSpec(memory_space=pltpu.VMEM),
        scratch_shapes=[
            pltpu.VMEM((TQ, D), jnp.float32),
            pltpu.VMEM((NQ, TQ, D), jnp.bfloat16),
            pltpu.VMEM((TQ, D), jnp.bfloat16),
            pltpu.SemaphoreType.DMA,
            pltpu.SemaphoreType.DMA((NC,)),
            pltpu.SemaphoreType.DMA((NC,)),
            pltpu.SemaphoreType.DMA((NC,)),
            pltpu.SemaphoreType.DMA((NC,)),
            pltpu.SemaphoreType.DMA((2 * NC,)),
            pltpu.SemaphoreType.DMA((2 * NC,)),
        ],
        compiler_params=pltpu.CompilerParams(collective_id=0),
    )(lids, mask, E)


# device time: 43016 ns/iter; 1.1734x vs baseline; 1.1734x over previous
import functools

import jax
import jax.numpy as jnp
from jax import lax
from jax.experimental import pallas as pl
from jax.experimental.pallas import tpu as pltpu

T = 2048
D = 1024
NQ = 4
TQ = T // NQ
NC = 4
CH = TQ // NC

_MESH = pl.DeviceIdType.MESH


def kernel(ids, E):
    Vs = E.shape[0]

    x = lax.axis_index("x")
    y = lax.axis_index("y")
    z = lax.axis_index("z")
    q = 2 * x + z
    vlo = y * Vs

    my_ids = lax.dynamic_slice(ids, (q * TQ,), (TQ,))
    in_range = jnp.logical_and(my_ids >= vlo, my_ids < vlo + Vs)
    lids = jnp.where(in_range, my_ids - vlo, 0).astype(jnp.int32)
    mask = in_range.astype(jnp.float32).reshape(TQ, 1)

    def body(lids_smem, mask_vmem, e_hbm, out_ref,
             gbuf, acc, ybuf, gsem, ys, yr, xs, xr, zs, zr, rs, rr):
        x = lax.axis_index("x")
        y = lax.axis_index("y")
        z = lax.axis_index("z")
        q = 2 * x + z
        qx = 2 * (1 - x) + z

        nbr_x = (1 - x, y, z)
        nbr_y = (x, 1 - y, z)
        nbr_z = (x, y, 1 - z)

        H = CH // 2
        for c in range(NC):
            for h in range(2):
                def issue(i, _, c=c, h=h):
                    t = c * CH + h * H + i * 8
                    for k in range(8):
                        pltpu.make_async_copy(
                            e_hbm.at[pl.ds(lids_smem[t + k], 1)],
                            gbuf.at[pl.ds(t + k, 1)],
                            gsem.at[2 * c + h]).start()
                    return 0
                lax.fori_loop(0, H // 8, issue, 0)

        bar = pltpu.get_barrier_semaphore()
        for nbr in (nbr_x, nbr_y, nbr_z):
            pl.semaphore_signal(bar, inc=1, device_id=nbr,
                                device_id_type=_MESH)
        pl.semaphore_wait(bar, 3)

        def drain_chunk(c):
            for h in range(2):
                def w(i, _, c=c, h=h):
                    for k in range(8):
                        pltpu.make_async_copy(e_hbm.at[pl.ds(0, 1)],
                                              gbuf.at[pl.ds(0, 1)],
                                              gsem.at[2 * c + h]).wait()
                    return 0
                lax.fori_loop(0, H // 8, w, 0)

        def rows(c):
            return pl.ds(c * CH, CH)

        def y_rdma(c):
            return pltpu.make_async_remote_copy(
                src_ref=acc.at[q, rows(c)], dst_ref=ybuf.at[rows(c)],
                send_sem=ys.at[c], recv_sem=yr.at[c],
                device_id=nbr_y, device_id_type=_MESH)

        def x_rdma(c):
            return pltpu.make_async_remote_copy(
                src_ref=acc.at[q, rows(c)], dst_ref=acc.at[q, rows(c)],
                send_sem=xs.at[c], recv_sem=xr.at[c],
                device_id=nbr_x, device_id_type=_MESH)

        def z_rdma(slot, c, si):
            return pltpu.make_async_remote_copy(
                src_ref=acc.at[slot, rows(c)], dst_ref=acc.at[slot, rows(c)],
                send_sem=zs.at[si], recv_sem=zr.at[si],
                device_id=nbr_z, device_id_type=_MESH)

        def dyn_rows(rstart):
            return pl.ds(rstart * CH, CH)

        def relay_rdma(slot, rstart, ri):
            return pltpu.make_async_remote_copy(
                src_ref=acc.at[slot, dyn_rows(rstart)],
                dst_ref=acc.at[slot, dyn_rows(rstart)],
                send_sem=rs.at[ri], recv_sem=rr.at[ri],
                device_id=nbr_y, device_id_type=_MESH)

        def wchunk(slot, cexpr):
            out_ref[pl.ds(slot * TQ + cexpr * CH, CH), :] = (
                acc[pl.ds(slot, 1), dyn_rows(cexpr)][0].astype(jnp.float32))

        for c in range(NC):
            drain_chunk(c)
            part = (gbuf[rows(c)] * mask_vmem[rows(c)]).astype(jnp.bfloat16)
            acc[pl.ds(q, 1), rows(c)] = part[None]
            y_rdma(c).start()

        def z_send(slot, c, si_base):
            if c == 0:
                @pl.when(y == 1)
                def _():
                    z_rdma(slot, 0, si_base).start()
            elif c == 1:
                @pl.when(y == 0)
                def _():
                    z_rdma(slot, 1, si_base).start()
            else:
                z_rdma(slot, c, si_base + c - 1).start()

        s1 = 2 * x + 1 - z
        s2 = 2 * (1 - x) + 1 - z
        rc = 1 - y
        mc = y

        for c in range(NC):
            y_rdma(c).wait_recv()
            acc[pl.ds(q, 1), rows(c)] = (
                acc[pl.ds(q, 1), rows(c)] + ybuf[rows(c)][None])
            x_rdma(c).start()
            z_send(q, c, 0)
            if c >= 1:
                x_rdma(c - 1).wait_recv()
                z_send(qx, c - 1, 3)
        x_rdma(NC - 1).wait_recv()
        z_send(qx, NC - 1, 3)

        z_rdma(s1, 0, 0).wait_recv()
        relay_rdma(s1, rc, 0).start()
        wchunk(s1, rc)
        z_rdma(s2, 0, 3).wait_recv()
        relay_rdma(s2, rc, 1).start()
        wchunk(s2, rc)

        out_ref[pl.ds(q * TQ, TQ), :] = acc[pl.ds(q, 1)][0].astype(jnp.float32)
        out_ref[pl.ds(qx * TQ, TQ), :] = acc[pl.ds(qx, 1)][0].astype(
            jnp.float32)

        z_rdma(s1, 2, 1).wait_recv()
        wchunk(s1, 2)
        z_rdma(s1, 3, 2).wait_recv()
        wchunk(s1, 3)
        z_rdma(s2, 2, 4).wait_recv()
        wchunk(s2, 2)
        z_rdma(s2, 3, 5).wait_recv()
        wchunk(s2, 3)

        relay_rdma(s1, mc, 0).wait_recv()
        wchunk(s1, mc)
        relay_rdma(s2, mc, 1).wait_recv()
        wchunk(s2, mc)

        for c in range(NC):
            y_rdma(c).wait_send()
            x_rdma(c).wait_send()
        for si in range(6):
            z_rdma(z, 0, si).wait_send()
        relay_rdma(s1, rc, 0).wait_send()
        relay_rdma(s2, rc, 1).wait_send()

        @functools.partial(pl.run_scoped, sem2=pltpu.SemaphoreType.REGULAR)
        def _(sem2):
            for nbr in (nbr_x, nbr_y, nbr_z):
                pl.semaphore_signal(sem2, inc=1, device_id=nbr,
                                    device_id_type=_MESH)
            pl.semaphore_wait(sem2, 3)

    return pl.pallas_call(
        body,
        out_shape=jax.ShapeDtypeStruct((T, D), jnp.float32),
        in_specs=[
            pl.BlockSpec(memory_space=pltpu.SMEM),
            pl.BlockSpec(memory_space=pltpu.VMEM),
            pl.BlockSpec(memory_space=pltpu.MemorySpace.HBM),
        ],
        out_specs=pl.BlockSpec(memory_space=pltpu.VMEM),
        scratch_shapes=[
            pltpu.VMEM((TQ, D), jnp.float32),
            pltpu.VMEM((NQ, TQ, D), jnp.bfloat16),
            pltpu.VMEM((TQ, D), jnp.bfloat16),
            pltpu.SemaphoreType.DMA((2 * NC,)),
            pltpu.SemaphoreType.DMA((NC,)),
            pltpu.SemaphoreType.DMA((NC,)),
            pltpu.SemaphoreType.DMA((NC,)),
            pltpu.SemaphoreType.DMA((NC,)),
            pltpu.SemaphoreType.DMA((6,)),
            pltpu.SemaphoreType.DMA((6,)),
            pltpu.SemaphoreType.DMA((2,)),
            pltpu.SemaphoreType.DMA((2,)),
        ],
        compiler_params=pltpu.CompilerParams(collective_id=0),
    )(lids, mask, E)
